# baseline (speedup 1.0000x reference)
import functools

import jax
import jax.numpy as jnp
from jax import lax
from jax.experimental import pallas as pl
from jax.experimental.pallas import tpu as pltpu
from jax.experimental.pallas import tpu_sc as plsc

N = 10000
E = 320000
D = 128
P = 10240
NC = 2
NS = 16
WIN = 112
NWT = 93
EPT = NWT * WIN
E_PAD = EPT * NC * NS
ROWS_PER_TILE = P // NS
NBUF = 3
BR = 1280

_mesh = plsc.VectorSubcoreMesh(core_axis_name="c", subcore_axis_name="s")
_sc_params = pltpu.CompilerParams(needs_layout_passes=False)


def _zero16():
    return jnp.zeros((16,), jnp.float32)


def _k1_body(dst_hbm, w_hbm, out_hbm, deg_sp, w_all, dw0, dw1, dw2, zbuf,
             de0, de1, de2, ds0, ds1, ds2):
    dstw = (dw0, dw1, dw2)
    dsem = (de0, de1, de2)
    ssem = (ds0, ds1, ds2)
    c = lax.axis_index("c")
    s = lax.axis_index("s")
    wid = c * NS + s
    ebase = wid * EPT

    for f in range(WIN // 16):
        zbuf[pl.ds(f * 16, 16)] = _zero16()
    r0 = s * ROWS_PER_TILE
    for t in range(5):
        pltpu.sync_copy(zbuf, deg_sp.at[pl.ds(r0 + t * WIN, WIN)])
    pltpu.sync_copy(zbuf.at[pl.ds(0, 80)], deg_sp.at[pl.ds(r0 + 560, 80)])

    pltpu.sync_copy(w_hbm.at[pl.ds(ebase, EPT)], w_all)
    plsc.subcore_barrier()

    for r in range(2):
        pltpu.async_copy(dst_hbm.at[pl.ds(ebase + r * WIN, WIN)],
                         dstw[r], dsem[r])

    def _body(i, r):
        pltpu.make_async_copy(dst_hbm.at[pl.ds(ebase + i * WIN, WIN)],
                              dstw[r], dsem[r]).wait()
        pltpu.async_copy(w_all.at[pl.ds(i * WIN, WIN)],
                         deg_sp.at[dstw[r]], ssem[r], add=True)
        rn = (r + 2) % NBUF

        @pl.when(i >= 1)
        def _retire():
            pltpu.make_async_copy(w_all.at[pl.ds((i - 1) * WIN, WIN)],
                                  deg_sp.at[dstw[rn]], ssem[rn]).wait()

        @pl.when(i + 2 < NWT)
        def _prefetch():
            pltpu.async_copy(dst_hbm.at[pl.ds(ebase + (i + 2) * WIN, WIN)],
                             dstw[rn], dsem[rn])

    @pl.loop(0, NWT // NBUF)
    def _outer(g):
        for r in range(NBUF):
            _body(g * NBUF + r, r)

    rl = (NWT - 1) % NBUF
    pltpu.make_async_copy(w_all.at[pl.ds((NWT - 1) * WIN, WIN)],
                          deg_sp.at[dstw[rl]], ssem[rl]).wait()

    plsc.subcore_barrier()
    pltpu.sync_copy(deg_sp.at[pl.ds(r0, ROWS_PER_TILE)],
                    out_hbm.at[c, pl.ds(r0, ROWS_PER_TILE)])


_k1 = pl.kernel(
    _k1_body,
    out_type=jax.ShapeDtypeStruct((NC, P), jnp.float32),
    mesh=_mesh,
    compiler_params=_sc_params,
    scratch_types=[
        pltpu.VMEM_SHARED((P,), jnp.float32),
        pltpu.VMEM((EPT,), jnp.float32),
        pltpu.VMEM((WIN,), jnp.int32),
        pltpu.VMEM((WIN,), jnp.int32),
        pltpu.VMEM((WIN,), jnp.int32),
        pltpu.VMEM((WIN,), jnp.float32),
        pltpu.SemaphoreType.DMA,
        pltpu.SemaphoreType.DMA,
        pltpu.SemaphoreType.DMA,
        pltpu.SemaphoreType.DMA,
        pltpu.SemaphoreType.DMA,
        pltpu.SemaphoreType.DMA,
    ],
)


def _k3_body(h_hbm, src_hbm, dst_hbm, w_hbm, out_hbm,
             acc_sp, r0_, r1_, r2_, s0_, s1_, s2_, d0, d1, d2, c0, c1, c2,
             ge0, ge1, ge2, gs0, gs1, gs2, se0, se1, se2):
    rows = (r0_, r1_, r2_)
    srcw = (s0_, s1_, s2_)
    dstw = (d0, d1, d2)
    cwin = (c0, c1, c2)
    gsem = (ge0, ge1, ge2)
    ssem = (gs0, gs1, gs2)
    esem = (se0, se1, se2)
    c = lax.axis_index("c")
    s = lax.axis_index("s")
    wid = c * NS + s
    ebase = wid * EPT

    @pl.loop(0, WIN)
    def _zero(j):
        for f in range(D // 16):
            r0_[j, pl.ds(f * 16, 16)] = _zero16()

    rr0 = s * ROWS_PER_TILE
    for t in range(5):
        pltpu.sync_copy(r0_, acc_sp.at[pl.ds(rr0 + t * WIN, WIN)])
    pltpu.sync_copy(r0_.at[pl.ds(0, 80)], acc_sp.at[pl.ds(rr0 + 560, 80)])
    plsc.subcore_barrier()

    def _fetch(k, b, sem):
        pltpu.async_copy(src_hbm.at[pl.ds(ebase + k * WIN, WIN)], srcw[b], sem)
        pltpu.async_copy(dst_hbm.at[pl.ds(ebase + k * WIN, WIN)], dstw[b], sem)
        pltpu.async_copy(w_hbm.at[pl.ds(ebase + k * WIN, WIN)], cwin[b], sem)

    def _fetch_wait(k, b, sem):
        pltpu.make_async_copy(src_hbm.at[pl.ds(ebase + k * WIN, WIN)],
                              srcw[b], sem).wait()
        pltpu.make_async_copy(dst_hbm.at[pl.ds(ebase + k * WIN, WIN)],
                              dstw[b], sem).wait()
        pltpu.make_async_copy(w_hbm.at[pl.ds(ebase + k * WIN, WIN)],
                              cwin[b], sem).wait()

    _fetch(0, 0, esem[0])
    _fetch(1, 1, esem[1])
    _fetch_wait(0, 0, esem[0])
    pltpu.async_copy(h_hbm.at[srcw[0]], rows[0], gsem[0])

    def _body(k, b):
        pltpu.make_async_copy(h_hbm.at[srcw[b]], rows[b], gsem[b]).wait()

        @pl.loop(0, WIN, unroll=4)
        def _row(j):
            idx = jnp.broadcast_to(j, (16,)).astype(jnp.int32)
            cb = plsc.load_gather(cwin[b], [idx])
            for f in range(D // 16):
                sl = pl.ds(f * 16, 16)
                rows[b][j, sl] = rows[b][j, sl] * cb

        pltpu.async_copy(rows[b], acc_sp.at[dstw[b]], ssem[b], add=True)

        b1 = (b + 2) % NBUF
        b2 = (b + 1) % NBUF

        @pl.when(k >= 1)
        def _retire():
            pltpu.make_async_copy(rows[b1], acc_sp.at[dstw[b1]],
                                  ssem[b1]).wait()

        @pl.when(k + 2 < NWT)
        def _prefetch():
            _fetch(k + 2, b1, esem[b1])

        @pl.when(k + 1 < NWT)
        def _next_gather():
            _fetch_wait(k + 1, b2, esem[b2])
            pltpu.async_copy(h_hbm.at[srcw[b2]], rows[b2], gsem[b2])

    @pl.loop(0, NWT // NBUF)
    def _outer(g):
        for b in range(NBUF):
            _body(g * NBUF + b, b)

    bl = (NWT - 1) % NBUF
    pltpu.make_async_copy(rows[bl], acc_sp.at[dstw[bl]], ssem[bl]).wait()

    plsc.subcore_barrier()
    for t in range(ROWS_PER_TILE // 128):
        rr = s * ROWS_PER_TILE + t * 128
        pltpu.sync_copy(acc_sp.at[pl.ds(rr, 128)], out_hbm.at[c, pl.ds(rr, 128)])


_k3 = pl.kernel(
    _k3_body,
    out_type=jax.ShapeDtypeStruct((NC, P, D), jnp.float32),
    mesh=_mesh,
    compiler_params=_sc_params,
    scratch_types=[
        pltpu.VMEM_SHARED((P, D), jnp.float32),
        pltpu.VMEM((WIN, D), jnp.float32),
        pltpu.VMEM((WIN, D), jnp.float32),
        pltpu.VMEM((WIN, D), jnp.float32),
        pltpu.VMEM((WIN,), jnp.int32),
        pltpu.VMEM((WIN,), jnp.int32),
        pltpu.VMEM((WIN,), jnp.int32),
        pltpu.VMEM((WIN,), jnp.int32),
        pltpu.VMEM((WIN,), jnp.int32),
        pltpu.VMEM((WIN,), jnp.int32),
        pltpu.VMEM((WIN,), jnp.float32),
        pltpu.VMEM((WIN,), jnp.float32),
        pltpu.VMEM((WIN,), jnp.float32),
        pltpu.SemaphoreType.DMA,
        pltpu.SemaphoreType.DMA,
        pltpu.SemaphoreType.DMA,
        pltpu.SemaphoreType.DMA,
        pltpu.SemaphoreType.DMA,
        pltpu.SemaphoreType.DMA,
        pltpu.SemaphoreType.DMA,
        pltpu.SemaphoreType.DMA,
        pltpu.SemaphoreType.DMA,
    ],
)


def _k2a_body(x_ref, w1_ref, h_ref):
    h_ref[...] = jnp.dot(x_ref[...], w1_ref[...],
                         preferred_element_type=jnp.float32)


def _k2b_body(degT_ref, h1_ref, dinv_ref, h1s_ref):
    d = lax.rsqrt(degT_ref[:, 0:1] + degT_ref[:, 1:2] + 1.0)
    dinv_ref[...] = d
    h1s_ref[...] = d * h1_ref[...]


def _k4_body(a_ref, h1s_ref, dinv_ref, b1_ref, g_ref, b_ref, w2_ref, h2s_ref):
    d = dinv_ref[...]
    z = d * (a_ref[0] + a_ref[1] + h1s_ref[...]) + b1_ref[...]
    z = jnp.maximum(z, 0.0)
    m = jnp.mean(z, axis=-1, keepdims=True)
    zc = z - m
    v = jnp.mean(zc * zc, axis=-1, keepdims=True)
    zn = g_ref[...] * zc * lax.rsqrt(v + 1e-5) + b_ref[...]
    h2s_ref[...] = d * jnp.dot(zn, w2_ref[...],
                               preferred_element_type=jnp.float32)


def _k5_body(a_ref, h2s_ref, dinv_ref, b2_ref, o_ref):
    d = dinv_ref[...]
    o_ref[...] = d * (a_ref[0] + a_ref[1] + h2s_ref[...]) + b2_ref[...]


def _blk(shape, imap):
    return pl.BlockSpec(shape, imap)


_row = lambda i: (i, 0)
_rep = lambda i: (0, 0)
_acc = lambda i: (0, i, 0)

_k2a = pl.pallas_call(
    _k2a_body,
    grid=(P // BR,),
    in_specs=[_blk((BR, D), _row), _blk((D, D), _rep)],
    out_specs=_blk((BR, D), _row),
    out_shape=jax.ShapeDtypeStruct((P, D), jnp.float32),
)

_k2b = pl.pallas_call(
    _k2b_body,
    grid=(P // BR,),
    in_specs=[_blk((BR, 2), _row), _blk((BR, D), _row)],
    out_specs=[_blk((BR, 1), _row), _blk((BR, D), _row)],
    out_shape=(jax.ShapeDtypeStruct((P, 1), jnp.float32),
               jax.ShapeDtypeStruct((P, D), jnp.float32)),
)

_k4 = pl.pallas_call(
    _k4_body,
    grid=(P // BR,),
    in_specs=[_blk((NC, BR, D), _acc), _blk((BR, D), _row),
              _blk((BR, 1), _row), _blk((1, D), _rep), _blk((1, D), _rep),
              _blk((1, D), _rep), _blk((D, D), _rep)],
    out_specs=_blk((BR, D), _row),
    out_shape=jax.ShapeDtypeStruct((P, D), jnp.float32),
)

_k5 = pl.pallas_call(
    _k5_body,
    grid=(P // BR,),
    in_specs=[_blk((NC, BR, D), _acc), _blk((BR, D), _row),
              _blk((BR, 1), _row), _blk((1, D), _rep)],
    out_specs=_blk((BR, D), _row),
    out_shape=jax.ShapeDtypeStruct((P, D), jnp.float32),
)


def kernel(x, edge_index, edge_weight, W1, b1, ln_gamma, ln_beta, W2, b2):
    pad = E_PAD - E
    fill = (jnp.arange(pad, dtype=jnp.int32) * 37) % N
    src = jnp.concatenate([edge_index[0], fill])
    dst = jnp.concatenate([edge_index[1], fill])
    w = jnp.concatenate([edge_weight, jnp.zeros((pad,), jnp.float32)])

    x_pad = jnp.zeros((P, D), jnp.float32).at[:N].set(x)

    deg2 = _k1(dst, w)
    h1 = _k2a(x_pad, W1)
    dinv, h1s = _k2b(deg2.T, h1)

    acc1 = _k3(h1s, src, dst, w)
    h2s = _k4(acc1, h1s, dinv,
              b1.reshape(1, D), ln_gamma.reshape(1, D), ln_beta.reshape(1, D),
              W2)
    acc2 = _k3(h2s, src, dst, w)
    out = _k5(acc2, h2s, dinv, b2.reshape(1, D))
    return out[:N]


# speedup vs baseline: 1.5287x; 1.5287x over previous
import functools

import jax
import jax.numpy as jnp
from jax import lax
from jax.experimental import pallas as pl
from jax.experimental.pallas import tpu as pltpu
from jax.experimental.pallas import tpu_sc as plsc

N = 10000
E = 320000
D = 128
P = 10240
NC = 2
NS = 16
WIN = 120
NWT = 84
EPT = NWT * WIN
E_PAD = EPT * NC * NS
ROWS_PER_TILE = P // NS
NBUF = 3
IBUF = 4
BR = 1280

_mesh = plsc.VectorSubcoreMesh(core_axis_name="c", subcore_axis_name="s")
_sc_params = pltpu.CompilerParams(needs_layout_passes=False)


def _zero16():
    return jnp.zeros((16,), jnp.float32)


def _k1_body(dst_hbm, w_hbm, out_hbm, deg_sp, w_all, dw0, dw1, dw2, zbuf,
             de0, de1, de2, ds0, ds1, ds2):
    dstw = (dw0, dw1, dw2)
    dsem = (de0, de1, de2)
    ssem = (ds0, ds1, ds2)
    c = lax.axis_index("c")
    s = lax.axis_index("s")
    wid = c * NS + s
    ebase = wid * EPT

    for f in range(WIN // 16):
        zbuf[pl.ds(f * 16, 16)] = _zero16()
    r0 = s * ROWS_PER_TILE
    for t in range(5):
        pltpu.sync_copy(zbuf, deg_sp.at[pl.ds(r0 + t * WIN, WIN)])
    pltpu.sync_copy(zbuf.at[pl.ds(0, 40)], deg_sp.at[pl.ds(r0 + 600, 40)])

    pltpu.sync_copy(w_hbm.at[pl.ds(ebase, EPT)], w_all)
    plsc.subcore_barrier()

    for r in range(2):
        pltpu.async_copy(dst_hbm.at[pl.ds(ebase + r * WIN, WIN)],
                         dstw[r], dsem[r])

    def _body(i, r):
        pltpu.make_async_copy(dst_hbm.at[pl.ds(ebase + i * WIN, WIN)],
                              dstw[r], dsem[r]).wait()
        pltpu.async_copy(w_all.at[pl.ds(i * WIN, WIN)],
                         deg_sp.at[dstw[r]], ssem[r], add=True)
        rn = (r + 2) % NBUF

        @pl.when(i >= 1)
        def _retire():
            pltpu.make_async_copy(w_all.at[pl.ds((i - 1) * WIN, WIN)],
                                  deg_sp.at[dstw[rn]], ssem[rn]).wait()

        @pl.when(i + 2 < NWT)
        def _prefetch():
            pltpu.async_copy(dst_hbm.at[pl.ds(ebase + (i + 2) * WIN, WIN)],
                             dstw[rn], dsem[rn])

    @pl.loop(0, NWT // NBUF)
    def _outer(g):
        for r in range(NBUF):
            _body(g * NBUF + r, r)

    rl = (NWT - 1) % NBUF
    pltpu.make_async_copy(w_all.at[pl.ds((NWT - 1) * WIN, WIN)],
                          deg_sp.at[dstw[rl]], ssem[rl]).wait()

    plsc.subcore_barrier()
    pltpu.sync_copy(deg_sp.at[pl.ds(r0, ROWS_PER_TILE)],
                    out_hbm.at[c, pl.ds(r0, ROWS_PER_TILE)])


_k1 = pl.kernel(
    _k1_body,
    out_type=jax.ShapeDtypeStruct((NC, P), jnp.float32),
    mesh=_mesh,
    compiler_params=_sc_params,
    scratch_types=[
        pltpu.VMEM_SHARED((P,), jnp.float32),
        pltpu.VMEM((EPT,), jnp.float32),
        pltpu.VMEM((WIN,), jnp.int32),
        pltpu.VMEM((WIN,), jnp.int32),
        pltpu.VMEM((WIN,), jnp.int32),
        pltpu.VMEM((WIN,), jnp.float32),
        pltpu.SemaphoreType.DMA,
        pltpu.SemaphoreType.DMA,
        pltpu.SemaphoreType.DMA,
        pltpu.SemaphoreType.DMA,
        pltpu.SemaphoreType.DMA,
        pltpu.SemaphoreType.DMA,
    ],
)


def _k3_body(h_hbm, src_hbm, dst_hbm, w_hbm, out_hbm,
             acc_sp, r0_, r1_, r2_, s0_, s1_, s2_, s3_, d0, d1, d2, d3,
             c0, c1, c2, c3,
             ge0, ge1, ge2, gs0, gs1, gs2, se0, se1, se2, se3):
    rows = (r0_, r1_, r2_)
    srcw = (s0_, s1_, s2_, s3_)
    dstw = (d0, d1, d2, d3)
    cwin = (c0, c1, c2, c3)
    gsem = (ge0, ge1, ge2)
    ssem = (gs0, gs1, gs2)
    esem = (se0, se1, se2, se3)
    c = lax.axis_index("c")
    s = lax.axis_index("s")
    wid = c * NS + s
    ebase = wid * EPT

    @pl.loop(0, WIN)
    def _zero(j):
        for f in range(D // 16):
            r0_[j, pl.ds(f * 16, 16)] = _zero16()

    rr0 = s * ROWS_PER_TILE
    for t in range(5):
        pltpu.sync_copy(r0_, acc_sp.at[pl.ds(rr0 + t * WIN, WIN)])
    pltpu.sync_copy(r0_.at[pl.ds(0, 40)], acc_sp.at[pl.ds(rr0 + 600, 40)])
    plsc.subcore_barrier()

    def _fetch(k, q):
        pltpu.async_copy(src_hbm.at[pl.ds(ebase + k * WIN, WIN)], srcw[q],
                         esem[q])
        pltpu.async_copy(dst_hbm.at[pl.ds(ebase + k * WIN, WIN)], dstw[q],
                         esem[q])
        pltpu.async_copy(w_hbm.at[pl.ds(ebase + k * WIN, WIN)], cwin[q],
                         esem[q])

    def _fetch_wait(k, q):
        pltpu.make_async_copy(src_hbm.at[pl.ds(ebase + k * WIN, WIN)],
                              srcw[q], esem[q]).wait()
        pltpu.make_async_copy(dst_hbm.at[pl.ds(ebase + k * WIN, WIN)],
                              dstw[q], esem[q]).wait()
        pltpu.make_async_copy(w_hbm.at[pl.ds(ebase + k * WIN, WIN)],
                              cwin[q], esem[q]).wait()

    for j in range(3):
        _fetch(j, j)
    for j in range(2):
        _fetch_wait(j, j)
        pltpu.async_copy(h_hbm.at[srcw[j]], rows[j], gsem[j])

    def _body(k, u):
        b = u % NBUF
        q = u % IBUF
        b1 = (u + 2) % NBUF
        q3 = (u + 3) % IBUF
        q2 = (u + 2) % IBUF

        pltpu.make_async_copy(h_hbm.at[srcw[q]], rows[b], gsem[b]).wait()

        @pl.loop(0, WIN, unroll=4)
        def _row(j):
            idx = jnp.broadcast_to(j, (16,)).astype(jnp.int32)
            cb = plsc.load_gather(cwin[q], [idx])
            for f in range(D // 16):
                sl = pl.ds(f * 16, 16)
                rows[b][j, sl] = rows[b][j, sl] * cb

        pltpu.async_copy(rows[b], acc_sp.at[dstw[q]], ssem[b], add=True)

        @pl.when(k >= 1)
        def _retire():
            qp = (u + 3) % IBUF
            pltpu.make_async_copy(rows[b1], acc_sp.at[dstw[qp]],
                                  ssem[b1]).wait()

        @pl.when(k + 3 < NWT)
        def _prefetch():
            _fetch(k + 3, q3)

        @pl.when(k + 2 < NWT)
        def _next_gather():
            _fetch_wait(k + 2, q2)
            pltpu.async_copy(h_hbm.at[srcw[q2]], rows[b1], gsem[b1])

    @pl.loop(0, NWT // 12)
    def _outer(g):
        for u in range(12):
            _body(g * 12 + u, u)

    pltpu.make_async_copy(rows[(NWT - 1) % NBUF],
                          acc_sp.at[dstw[(NWT - 1) % IBUF]],
                          ssem[(NWT - 1) % NBUF]).wait()

    plsc.subcore_barrier()
    for t in range(ROWS_PER_TILE // 128):
        rr = s * ROWS_PER_TILE + t * 128
        pltpu.sync_copy(acc_sp.at[pl.ds(rr, 128)], out_hbm.at[c, pl.ds(rr, 128)])


_k3 = pl.kernel(
    _k3_body,
    out_type=jax.ShapeDtypeStruct((NC, P, D), jnp.float32),
    mesh=_mesh,
    compiler_params=_sc_params,
    scratch_types=[
        pltpu.VMEM_SHARED((P, D), jnp.float32),
        pltpu.VMEM((WIN, D), jnp.float32),
        pltpu.VMEM((WIN, D), jnp.float32),
        pltpu.VMEM((WIN, D), jnp.float32),
        pltpu.VMEM((WIN,), jnp.int32),
        pltpu.VMEM((WIN,), jnp.int32),
        pltpu.VMEM((WIN,), jnp.int32),
        pltpu.VMEM((WIN,), jnp.int32),
        pltpu.VMEM((WIN,), jnp.int32),
        pltpu.VMEM((WIN,), jnp.int32),
        pltpu.VMEM((WIN,), jnp.int32),
        pltpu.VMEM((WIN,), jnp.int32),
        pltpu.VMEM((WIN,), jnp.float32),
        pltpu.VMEM((WIN,), jnp.float32),
        pltpu.VMEM((WIN,), jnp.float32),
        pltpu.VMEM((WIN,), jnp.float32),
        pltpu.SemaphoreType.DMA,
        pltpu.SemaphoreType.DMA,
        pltpu.SemaphoreType.DMA,
        pltpu.SemaphoreType.DMA,
        pltpu.SemaphoreType.DMA,
        pltpu.SemaphoreType.DMA,
        pltpu.SemaphoreType.DMA,
        pltpu.SemaphoreType.DMA,
        pltpu.SemaphoreType.DMA,
        pltpu.SemaphoreType.DMA,
    ],
)


def _k2a_body(x_ref, w1_ref, h_ref):
    h_ref[...] = jnp.dot(x_ref[...], w1_ref[...],
                         preferred_element_type=jnp.float32)


def _k2b_body(degT_ref, h1_ref, dinv_ref, h1s_ref):
    d = lax.rsqrt(degT_ref[:, 0:1] + degT_ref[:, 1:2] + 1.0)
    dinv_ref[...] = d
    h1s_ref[...] = d * h1_ref[...]


def _k4_body(a_ref, h1s_ref, dinv_ref, b1_ref, g_ref, b_ref, w2_ref, h2s_ref):
    d = dinv_ref[...]
    z = d * (a_ref[0] + a_ref[1] + h1s_ref[...]) + b1_ref[...]
    z = jnp.maximum(z, 0.0)
    m = jnp.mean(z, axis=-1, keepdims=True)
    zc = z - m
    v = jnp.mean(zc * zc, axis=-1, keepdims=True)
    zn = g_ref[...] * zc * lax.rsqrt(v + 1e-5) + b_ref[...]
    h2s_ref[...] = d * jnp.dot(zn, w2_ref[...],
                               preferred_element_type=jnp.float32)


def _k5_body(a_ref, h2s_ref, dinv_ref, b2_ref, o_ref):
    d = dinv_ref[...]
    o_ref[...] = d * (a_ref[0] + a_ref[1] + h2s_ref[...]) + b2_ref[...]


def _blk(shape, imap):
    return pl.BlockSpec(shape, imap)


_row = lambda i: (i, 0)
_rep = lambda i: (0, 0)
_acc = lambda i: (0, i, 0)

_k2a = pl.pallas_call(
    _k2a_body,
    grid=(P // BR,),
    in_specs=[_blk((BR, D), _row), _blk((D, D), _rep)],
    out_specs=_blk((BR, D), _row),
    out_shape=jax.ShapeDtypeStruct((P, D), jnp.float32),
)

_k2b = pl.pallas_call(
    _k2b_body,
    grid=(P // BR,),
    in_specs=[_blk((BR, 2), _row), _blk((BR, D), _row)],
    out_specs=[_blk((BR, 1), _row), _blk((BR, D), _row)],
    out_shape=(jax.ShapeDtypeStruct((P, 1), jnp.float32),
               jax.ShapeDtypeStruct((P, D), jnp.float32)),
)

_k4 = pl.pallas_call(
    _k4_body,
    grid=(P // BR,),
    in_specs=[_blk((NC, BR, D), _acc), _blk((BR, D), _row),
              _blk((BR, 1), _row), _blk((1, D), _rep), _blk((1, D), _rep),
              _blk((1, D), _rep), _blk((D, D), _rep)],
    out_specs=_blk((BR, D), _row),
    out_shape=jax.ShapeDtypeStruct((P, D), jnp.float32),
)

_k5 = pl.pallas_call(
    _k5_body,
    grid=(P // BR,),
    in_specs=[_blk((NC, BR, D), _acc), _blk((BR, D), _row),
              _blk((BR, 1), _row), _blk((1, D), _rep)],
    out_specs=_blk((BR, D), _row),
    out_shape=jax.ShapeDtypeStruct((P, D), jnp.float32),
)


def kernel(x, edge_index, edge_weight, W1, b1, ln_gamma, ln_beta, W2, b2):
    pad = E_PAD - E
    fill = (jnp.arange(pad, dtype=jnp.int32) * 37) % N
    src = jnp.concatenate([edge_index[0], fill])
    dst = jnp.concatenate([edge_index[1], fill])
    w = jnp.concatenate([edge_weight, jnp.zeros((pad,), jnp.float32)])

    x_pad = jnp.zeros((P, D), jnp.float32).at[:N].set(x)

    deg2 = _k1(dst, w)
    h1 = _k2a(x_pad, W1)
    dinv, h1s = _k2b(deg2.T, h1)

    acc1 = _k3(h1s, src, dst, w)
    h2s = _k4(acc1, h1s, dinv,
              b1.reshape(1, D), ln_gamma.reshape(1, D), ln_beta.reshape(1, D),
              W2)
    acc2 = _k3(h2s, src, dst, w)
    out = _k5(acc2, h2s, dinv, b2.reshape(1, D))
    return out[:N]
